# baseline (speedup 1.0000x reference)
import dataclasses
import functools

import jax
import jax.numpy as jnp
from jax import lax
from jax.experimental import pallas as pl
from jax.experimental.pallas import tpu as pltpu
from jax.experimental.pallas import tpu_sc as plsc

_NUM_CLASSES = 1000
_ROWS = 16384
_NUM_WORKERS = 32
_ROWS_PER_WORKER = _ROWS // _NUM_WORKERS
_BLOCK = 32
_NBLK = _ROWS_PER_WORKER // _BLOCK
_LANES = 16


def _sc_one_hot(x_hbm, out_hbm, idx_v, z0, z1, sem0, sem1):
    wid = lax.axis_index("s") * 2 + lax.axis_index("c")
    base = wid * _ROWS_PER_WORKER

    pltpu.sync_copy(x_hbm.at[pl.ds(base, _ROWS_PER_WORKER)], idx_v)

    rows16 = lax.iota(jnp.int32, _LANES)
    ones16 = jnp.full((_LANES,), 1, jnp.int32)
    zeros16 = jnp.zeros((_LANES,), jnp.int32)

    bufs = (z0, z1)
    sems = (sem0, sem1)

    for z in bufs:
        @pl.loop(0, _BLOCK)
        def _(r, z=z):
            @pl.loop(0, _NUM_CLASSES - _LANES, step=_LANES)
            def _(c, z=z, r=r):
                z[r, pl.ds(c, _LANES)] = zeros16

            z[r, pl.ds(_NUM_CLASSES - _LANES, _LANES)] = zeros16

    def set_ones(jb, z, val):
        for h in range(_BLOCK // _LANES):
            idx16 = idx_v[pl.ds(jb * _BLOCK + h * _LANES, _LANES)]
            plsc.store_scatter(z, [rows16 + (h * _LANES), idx16], val)

    def fire(jb, z, sem):
        pltpu.make_async_copy(
            z, out_hbm.at[pl.ds(base + jb * _BLOCK, _BLOCK)], sem).start()

    def drain(z, sem):
        pltpu.make_async_copy(
            z, out_hbm.at[pl.ds(base, _BLOCK)], sem).wait()

    set_ones(0, z0, ones16)
    fire(0, z0, sem0)
    set_ones(1, z1, ones16)
    fire(1, z1, sem1)

    @pl.loop(0, _NBLK - 2, step=2)
    def _(i):
        drain(z0, sem0)
        set_ones(i, z0, zeros16)
        set_ones(i + 2, z0, ones16)
        fire(i + 2, z0, sem0)

        drain(z1, sem1)
        set_ones(i + 1, z1, zeros16)
        set_ones(i + 3, z1, ones16)
        fire(i + 3, z1, sem1)

    drain(z0, sem0)
    drain(z1, sem1)


def kernel(x):
    idx = x.astype(jnp.int32).reshape(_ROWS)
    mesh = plsc.VectorSubcoreMesh(core_axis_name="c", subcore_axis_name="s")
    cp = pltpu.CompilerParams()
    if "needs_layout_passes" in pltpu.CompilerParams.__dataclass_fields__:
        cp = dataclasses.replace(cp, needs_layout_passes=False)
    if "use_tc_tiling_on_sc" in pltpu.CompilerParams.__dataclass_fields__:
        cp = dataclasses.replace(cp, use_tc_tiling_on_sc=False)
    sc_kernel = pl.kernel(
        _sc_one_hot,
        out_type=jax.ShapeDtypeStruct((_ROWS, _NUM_CLASSES), jnp.int32),
        mesh=mesh,
        scratch_types=[
            pltpu.VMEM((_ROWS_PER_WORKER,), jnp.int32),
            pltpu.VMEM((_BLOCK, _NUM_CLASSES), jnp.int32),
            pltpu.VMEM((_BLOCK, _NUM_CLASSES), jnp.int32),
            pltpu.SemaphoreType.DMA,
            pltpu.SemaphoreType.DMA,
        ],
        compiler_params=cp,
    )
    return sc_kernel(idx)


# speedup vs baseline: 1.4502x; 1.4502x over previous
import dataclasses
import functools

import jax
import jax.numpy as jnp
from jax import lax
from jax.experimental import pallas as pl
from jax.experimental.pallas import tpu as pltpu
from jax.experimental.pallas import tpu_sc as plsc

_NUM_CLASSES = 1000
_ROWS = 16384
_NUM_WORKERS = 32
_ROWS_PER_WORKER = _ROWS // _NUM_WORKERS
_BLOCK = 32
_NBUF = 3
_NBLK = _ROWS_PER_WORKER // _BLOCK
_LANES = 16


def _sc_one_hot(x_hbm, out_hbm, idx_v, z0, z1, z2, sem0, sem1, sem2):
    wid = lax.axis_index("s") * 2 + lax.axis_index("c")
    base = wid * _ROWS_PER_WORKER

    pltpu.sync_copy(x_hbm.at[pl.ds(base, _ROWS_PER_WORKER)], idx_v)

    rows16 = lax.iota(jnp.int32, _LANES)
    ones16 = jnp.full((_LANES,), 1, jnp.int32)
    zeros16 = jnp.zeros((_LANES,), jnp.int32)

    bufs = (z0, z1, z2)
    sems = (sem0, sem1, sem2)

    for z in bufs:
        @pl.loop(0, _BLOCK)
        def _(r, z=z):
            @pl.loop(0, _NUM_CLASSES - _LANES, step=_LANES)
            def _(c, z=z, r=r):
                z[r, pl.ds(c, _LANES)] = zeros16

            z[r, pl.ds(_NUM_CLASSES - _LANES, _LANES)] = zeros16

    def set_ones(jb, z, val):
        for h in range(_BLOCK // _LANES):
            idx16 = idx_v[pl.ds(jb * _BLOCK + h * _LANES, _LANES)]
            plsc.store_scatter(z, [rows16 + (h * _LANES), idx16], val)

    def fire(jb, z, sem):
        pltpu.make_async_copy(
            z, out_hbm.at[pl.ds(base + jb * _BLOCK, _BLOCK)], sem).start()

    def drain(z, sem):
        pltpu.make_async_copy(
            z, out_hbm.at[pl.ds(base, _BLOCK)], sem).wait()

    for jb in range(_NBLK):
        b = jb % _NBUF
        if jb >= _NBUF:
            drain(bufs[b], sems[b])
            set_ones(jb - _NBUF, bufs[b], zeros16)
        set_ones(jb, bufs[b], ones16)
        fire(jb, bufs[b], sems[b])

    for b in range(_NBUF):
        drain(bufs[b], sems[b])


def kernel(x):
    idx = x.astype(jnp.int32).reshape(_ROWS)
    mesh = plsc.VectorSubcoreMesh(core_axis_name="c", subcore_axis_name="s")
    cp = pltpu.CompilerParams()
    if "needs_layout_passes" in pltpu.CompilerParams.__dataclass_fields__:
        cp = dataclasses.replace(cp, needs_layout_passes=False)
    sc_kernel = pl.kernel(
        _sc_one_hot,
        out_type=jax.ShapeDtypeStruct((_ROWS, _NUM_CLASSES), jnp.int32),
        mesh=mesh,
        scratch_types=[
            pltpu.VMEM((_ROWS_PER_WORKER,), jnp.int32),
            pltpu.VMEM((_BLOCK, _NUM_CLASSES), jnp.int32),
            pltpu.VMEM((_BLOCK, _NUM_CLASSES), jnp.int32),
            pltpu.VMEM((_BLOCK, _NUM_CLASSES), jnp.int32),
            pltpu.SemaphoreType.DMA,
            pltpu.SemaphoreType.DMA,
            pltpu.SemaphoreType.DMA,
        ],
        compiler_params=cp,
    )
    return sc_kernel(idx)


# speedup vs baseline: 1.8755x; 1.2933x over previous
import jax
import jax.numpy as jnp
from jax.experimental import pallas as pl
from jax.experimental.pallas import tpu as pltpu

_NUM_CLASSES = 1000
_PADDED = 1024
_ROWS = 16384
_BLOCK_ROWS = 1024


def _one_hot_block(x_ref, o_ref):
    idx = x_ref[:, 0]
    iota = jax.lax.broadcasted_iota(jnp.int32, (_BLOCK_ROWS, _PADDED), 1)
    o_ref[...] = (idx[:, None] == iota).astype(jnp.int32)


def kernel(x):
    idx = x.astype(jnp.int32)
    padded = pl.pallas_call(
        _one_hot_block,
        grid=(_ROWS // _BLOCK_ROWS,),
        in_specs=[pl.BlockSpec((_BLOCK_ROWS, 1), lambda i: (i, 0))],
        out_specs=pl.BlockSpec((_BLOCK_ROWS, _PADDED), lambda i: (i, 0)),
        out_shape=jax.ShapeDtypeStruct((_ROWS, _PADDED), jnp.int32),
        compiler_params=pltpu.CompilerParams(
            dimension_semantics=("parallel",)),
    )(idx)
    return padded[:, :_NUM_CLASSES]
